# baseline (speedup 1.0000x reference)
import jax
import jax.numpy as jnp
from jax.experimental import pallas as pl
from jax.experimental.pallas import tpu as pltpu

B, N, D = 4096, 512, 16
VEL_W, SUR_W = 0.6, 0.4
BR = 16
STEPS = B // BR
SBR = 256
SSTEPS = B // SBR


def _salience_body(zt_ref, ztm_ref, p_ref, out_ref, idx_ref, acc_ref):
    i = pl.program_id(0)
    zt = zt_ref[...]
    out_ref[...] = zt
    ztm = ztm_ref[...]
    p = p_ref[...]

    diff = zt - ztm
    vel2 = jnp.sum(diff * diff, axis=-1)
    zdot = jnp.sum(zt * p[None], axis=-1)
    nx2 = jnp.sum(zt * zt, axis=-1)
    ny2 = jnp.sum(p * p, axis=-1)[None]

    vel = jnp.sqrt(vel2)
    nx = jnp.sqrt(nx2)
    ny = jnp.sqrt(ny2)
    cos = zdot / jnp.maximum(nx * ny, 1e-8)
    surprise = jnp.clip(1.0 - cos, 0.0, 2.0) / 2.0
    sal = VEL_W * vel + SUR_W * surprise

    mn = jnp.min(sal, axis=-1, keepdims=True)
    mx = jnp.max(sal, axis=-1, keepdims=True)
    saln = (sal - mn) / (mx - mn + 1e-8)
    bsum = jnp.sum(saln, axis=0, keepdims=True)

    @pl.when(i == 0)
    def _init():
        acc_ref[...] = bsum

    @pl.when(i != 0)
    def _accum():
        acc_ref[...] = acc_ref[...] + bsum

    @pl.when(i == STEPS - 1)
    def _finish():
        acc = acc_ref[...]
        m = jnp.max(acc)
        eid = jax.lax.broadcasted_iota(jnp.int32, (1, N), 1)
        idx_ref[0, 0] = jnp.min(jnp.where(acc == m, eid, jnp.int32(2**30)))


def _scatter_body(idx_ref, mt_ref, y_ref, o_ref):
    sub = idx_ref[0] % 8
    ent = jax.lax.broadcasted_iota(jnp.int32, (SBR, 8, D), 1)
    o_ref[...] = jnp.where(ent == sub, mt_ref[...], y_ref[...])


def kernel(z_t, z_tm1, prior, mask_token):
    out_copy, idx = pl.pallas_call(
        _salience_body,
        grid=(STEPS,),
        in_specs=[
            pl.BlockSpec((BR, N, D), lambda i: (i, 0, 0)),
            pl.BlockSpec((BR, N, D), lambda i: (i, 0, 0)),
            pl.BlockSpec((N, D), lambda i: (0, 0)),
        ],
        out_specs=[
            pl.BlockSpec((BR, N, D), lambda i: (i, 0, 0)),
            pl.BlockSpec(memory_space=pltpu.SMEM),
        ],
        out_shape=[
            jax.ShapeDtypeStruct((B, N, D), jnp.float32),
            jax.ShapeDtypeStruct((1, 1), jnp.int32),
        ],
        scratch_shapes=[pltpu.VMEM((1, N), jnp.float32)],
    )(z_t, z_tm1, prior)

    mt3 = mask_token.reshape(1, 1, D)
    idx_flat = idx.reshape((1,))

    masked = pl.pallas_call(
        _scatter_body,
        grid_spec=pltpu.PrefetchScalarGridSpec(
            num_scalar_prefetch=1,
            grid=(SSTEPS,),
            in_specs=[
                pl.BlockSpec((1, 1, D), lambda i, sref: (0, 0, 0)),
                pl.BlockSpec((SBR, 8, D), lambda i, sref: (i, sref[0] // 8, 0)),
            ],
            out_specs=pl.BlockSpec((SBR, 8, D), lambda i, sref: (i, sref[0] // 8, 0)),
        ),
        out_shape=jax.ShapeDtypeStruct((B, N, D), jnp.float32),
        input_output_aliases={2: 0},
    )(idx_flat, mt3, out_copy)

    return masked
